# Initial kernel scaffold; baseline (speedup 1.0000x reference)
import jax
import jax.numpy as jnp
from jax.experimental import pallas as pl


def kernel(x, w, b, caw1, cab1, caw2, cab2, tw, tb, gamma):
    raise NotImplementedError("write your pallas kernel here")


# speedup vs baseline: 2.8966x; 2.8966x over previous
import functools

import jax
import jax.numpy as jnp
from jax.experimental import pallas as pl
from jax.experimental.pallas import tpu as pltpu

P = 8
C = 32
LANES = P * C
W0 = 8


def _pack_conv(w, dtype):
    D = w.shape[0]
    S = w.shape[-1] // C
    w7 = w.reshape(D, 3, 3, C, S, C)
    w7 = jnp.transpose(w7, (0, 4, 1, 2, 3, 5))
    pi = jnp.arange(P)[:, None]
    po = jnp.arange(P)[None, :]
    sel = jnp.stack([(pi - po + 1 == t) for t in range(3)]).astype(w.dtype)
    main = jnp.einsum('tpq,dsytcf->dsypcqf', sel, w7)
    main = main.reshape(D, S, 3, P * C, P * C)
    zed = jnp.zeros((D, S, 3, C, P, C), w.dtype)
    left = zed.at[..., 0, :].set(w7[:, :, :, 0]).reshape(D, S, 3, C, P * C)
    right = zed.at[..., P - 1, :].set(w7[:, :, :, 2]).reshape(D, S, 3, C, P * C)
    return main.astype(dtype), left.astype(dtype), right.astype(dtype)


def _dfag_kernel(x_ref, wm_ref, wl_ref, wr_ref, bt_ref,
                 caw1_ref, cab1_ref, caw2_ref, cab2_ref,
                 tm_ref, tl_ref, tr_ref, tbt_ref, gamma_ref,
                 o_ref, pad_ref, *, H, W8, CR, n_dfa):
    HG = H * W8
    HW = H * W8 * P

    pad_ref[...] = jnp.zeros_like(pad_ref)

    cdt = pad_ref.dtype

    def conv(x_flat, mats, biases, relus):
        pad_ref[1:H + 1, W0:W0 + W8, :] = (
            x_flat.reshape(H, W8, LANES).astype(cdt))
        nout = len(mats)
        accs = [None] * nout
        for ky in range(3):
            pm = pad_ref[ky:ky + H, W0:W0 + W8, :].reshape(HG, LANES)
            plft = pad_ref[ky:ky + H, W0 - 1:W0 - 1 + W8,
                           LANES - C:].reshape(HG, C)
            prgt = pad_ref[ky:ky + H, W0 + 1:W0 + 1 + W8, :C].reshape(HG, C)
            for t, (m3, l3, r3) in enumerate(mats):
                a = jnp.dot(pm, m3[ky], preferred_element_type=jnp.float32)
                a = a + jnp.dot(plft, l3[ky],
                                preferred_element_type=jnp.float32)
                a = a + jnp.dot(prgt, r3[ky],
                                preferred_element_type=jnp.float32)
                accs[t] = a if accs[t] is None else accs[t] + a
        outs = []
        for acc, bias, relu in zip(accs, biases, relus):
            if bias is not None:
                acc = acc + bias
            if relu:
                acc = jnp.maximum(acc, 0.0)
            outs.append(acc)
        return outs

    def dfa_mats(d, s):
        return ([wm_ref[d, s, ky] for ky in range(3)],
                [wl_ref[d, s, ky] for ky in range(3)],
                [wr_ref[d, s, ky] for ky in range(3)])

    def fold32(v, op):
        v = op(v[:, :128], v[:, 128:])
        v = op(v[:, :64], v[:, 64:])
        return op(v[:, :32], v[:, 32:])

    def tile8(v):
        return jnp.concatenate([v] * P, axis=-1)

    def ca_layer(x_flat, d, r):
        pooled = jnp.sum(x_flat, axis=0, keepdims=True) * (1.0 / HW)
        pooled = fold32(pooled, jnp.add)
        w1 = caw1_ref[d][:, r * CR:(r + 1) * CR]
        b1 = cab1_ref[d][:, r * CR:(r + 1) * CR]
        w2 = caw2_ref[d][:, r * C:(r + 1) * C]
        b2 = cab2_ref[d][:, r * C:(r + 1) * C]
        h = jnp.maximum(
            jnp.dot(pooled, w1, preferred_element_type=jnp.float32) + b1, 0.0)
        y = jax.nn.sigmoid(
            jnp.dot(h, w2, preferred_element_type=jnp.float32) + b2)
        return x_flat * tile8(y)

    def rcab(x_flat, d, r):
        s1, s2 = 2 * r, 2 * r + 1
        (h1,) = conv(x_flat, [dfa_mats(d, s1)], [bt_ref[d, s1]], [True])
        (h2,) = conv(h1, [dfa_mats(d, s2)], [bt_ref[d, s2]], [False])
        return ca_layer(h2, d, r) + x_flat

    def dfa_block(d, x_flat):
        x_flat = rcab(x_flat, d, 0)
        x_flat = rcab(x_flat, d, 1)
        q, k = conv(x_flat, [dfa_mats(d, 4), dfa_mats(d, 5)],
                    [None, None], [False, False])
        (v,) = conv(k, [dfa_mats(d, 6)], [None], [False])
        e3 = (q * k).reshape(H, W8, LANES)
        m = fold32(jnp.max(e3, axis=1), jnp.maximum)
        e = jnp.exp(e3 - tile8(m)[:, None, :])
        s = fold32(jnp.sum(e, axis=1), jnp.add)
        attn = e * pl.reciprocal(tile8(s), approx=False)[:, None, :]
        out = gamma_ref[d] * (v.reshape(H, W8, LANES) * attn)
        return out.reshape(HG, LANES)

    x0 = x_ref[0].reshape(HG, LANES)
    res = jax.lax.fori_loop(0, n_dfa, dfa_block, x0)
    tmats = ([tm_ref[0, 0, ky] for ky in range(3)],
             [tl_ref[0, 0, ky] for ky in range(3)],
             [tr_ref[0, 0, ky] for ky in range(3)])
    (tail,) = conv(res, [tmats], [tbt_ref[...]], [False])
    o_ref[0] = (tail + x0).reshape(H, W8, LANES).astype(o_ref.dtype)


def kernel(x, w, b, caw1, cab1, caw2, cab2, tw, tb, gamma):
    N, H, W, C_ = x.shape
    assert C_ == C and W % P == 0
    W8 = W // P
    n_dfa = w.shape[0]
    CR = caw1.shape[-1] // 2
    cdt = jnp.bfloat16

    wm, wl, wr = _pack_conv(w, cdt)
    tm, tl, tr = _pack_conv(tw[None, ...], cdt)
    bt = jnp.tile(b.reshape(n_dfa, 7, 1, 1, C)[:, :4], (1, 1, 1, P, 1))
    bt = bt.reshape(n_dfa, 4, 1, LANES)
    tbt = jnp.tile(tb, (1, P))

    xp = x.reshape(N, H, W8, LANES)

    def full_spec(a):
        nd = a.ndim
        return pl.BlockSpec(a.shape, lambda n: (0,) * nd)

    _body = functools.partial(_dfag_kernel, H=H, W8=W8, CR=CR, n_dfa=n_dfa)

    out = pl.pallas_call(
        _body,
        out_shape=jax.ShapeDtypeStruct((N, H, W8, LANES), x.dtype),
        grid=(N,),
        in_specs=[
            pl.BlockSpec((1, H, W8, LANES), lambda n: (n, 0, 0, 0)),
            full_spec(wm), full_spec(wl), full_spec(wr), full_spec(bt),
            full_spec(caw1), full_spec(cab1), full_spec(caw2),
            full_spec(cab2),
            full_spec(tm), full_spec(tl), full_spec(tr), full_spec(tbt),
            pl.BlockSpec(memory_space=pltpu.MemorySpace.SMEM),
        ],
        out_specs=pl.BlockSpec((1, H, W8, LANES), lambda n: (n, 0, 0, 0)),
        scratch_shapes=[pltpu.VMEM((H + 2, W0 + W8 + 2, LANES), cdt)],
        compiler_params=pltpu.CompilerParams(
            dimension_semantics=("parallel",)),
    )(xp, wm, wl, wr, bt, caw1, cab1, caw2, cab2, tm, tl, tr, tbt, gamma)
    return out.reshape(N, H, W, C)


# speedup vs baseline: 3.8439x; 1.3271x over previous
import functools

import jax
import jax.numpy as jnp
from jax.experimental import pallas as pl
from jax.experimental.pallas import tpu as pltpu

P = 8
C = 32
LANES = P * C
W0 = 8


def _pack_conv(w, dtype):
    D = w.shape[0]
    S = w.shape[-1] // C
    w7 = w.reshape(D, 3, 3, C, S, C)
    w7 = jnp.transpose(w7, (0, 4, 1, 2, 3, 5))
    pi = jnp.arange(P)[:, None]
    po = jnp.arange(P)[None, :]
    sel = jnp.stack([(pi - po + 1 == t) for t in range(3)]).astype(w.dtype)
    main = jnp.einsum('tpq,dsytcf->dsypcqf', sel, w7)
    main = main.reshape(D, S, 3, P * C, P * C)
    zed = jnp.zeros((D, S, 3, C, P, C), w.dtype)
    left = zed.at[..., 0, :].set(w7[:, :, :, 0]).reshape(D, S, 3, C, P * C)
    right = zed.at[..., P - 1, :].set(w7[:, :, :, 2]).reshape(D, S, 3, C, P * C)
    edge = jnp.concatenate([left, right], axis=-2)
    return main.astype(dtype), edge.astype(dtype)


def _dfag_kernel(x_ref, wm_ref, we_ref, bt_ref,
                 caw1t_ref, cab1_ref, caw2t_ref, cab2t_ref, afold_ref,
                 tm_ref, te_ref, tbt_ref, gamma_ref,
                 o_ref, pad_ref, ec_ref, *, H, W8, CR, n_dfa):
    HG = H * W8
    HW = H * W8 * P

    pad_ref[...] = jnp.zeros_like(pad_ref)
    ec_ref[...] = jnp.zeros_like(ec_ref)

    cdt = pad_ref.dtype

    def conv(x_flat, mats, biases, relus):
        x3 = x_flat.reshape(H, W8, LANES).astype(cdt)
        pad_ref[1:H + 1, W0:W0 + W8, :] = x3
        ec_ref[1:H + 1, W0 + 1:W0 + 1 + W8, :C] = x3[:, :, LANES - C:]
        ec_ref[1:H + 1, W0 - 1:W0 - 1 + W8, C:] = x3[:, :, :C]
        nout = len(mats)
        accs = [None] * nout
        for ky in range(3):
            pm = pad_ref[ky:ky + H, W0:W0 + W8, :].reshape(HG, LANES)
            pe = ec_ref[ky:ky + H, W0:W0 + W8, :].reshape(HG, 2 * C)
            for t, (m3, e3) in enumerate(mats):
                a = jnp.dot(pm, m3[ky], preferred_element_type=jnp.float32)
                a = a + jnp.dot(pe, e3[ky],
                                preferred_element_type=jnp.float32)
                accs[t] = a if accs[t] is None else accs[t] + a
        outs = []
        for acc, bias, relu in zip(accs, biases, relus):
            if bias is not None:
                acc = acc + bias
            if relu:
                acc = jnp.maximum(acc, 0.0)
            outs.append(acc)
        return outs

    def dfa_mats(d, s):
        return ([wm_ref[d, s, ky] for ky in range(3)],
                [we_ref[d, s, ky] for ky in range(3)])

    def fold32(v, op):
        v = op(v[:, :128], v[:, 128:])
        v = op(v[:, :64], v[:, 64:])
        return op(v[:, :32], v[:, 32:])

    def tile8(v):
        return jnp.concatenate([v] * P, axis=-1)

    def ca_layer(x_flat, d, r):
        pooled = jnp.sum(x_flat, axis=0, keepdims=True) * (1.0 / HW)
        w1 = caw1t_ref[d][:, r * CR:(r + 1) * CR]
        b1 = cab1_ref[d][:, r * CR:(r + 1) * CR]
        h = jnp.maximum(
            jnp.dot(pooled, w1, preferred_element_type=jnp.float32) + b1, 0.0)
        y = jax.nn.sigmoid(
            jnp.dot(h, caw2t_ref[d, r], preferred_element_type=jnp.float32)
            + cab2t_ref[d, r])
        return x_flat * y

    def rcab(x_flat, d, r):
        s1, s2 = 2 * r, 2 * r + 1
        (h1,) = conv(x_flat, [dfa_mats(d, s1)], [bt_ref[d, s1]], [True])
        (h2,) = conv(h1, [dfa_mats(d, s2)], [bt_ref[d, s2]], [False])
        return ca_layer(h2, d, r) + x_flat

    def dfa_block(d, x_flat):
        x_flat = rcab(x_flat, d, 0)
        x_flat = rcab(x_flat, d, 1)
        q, k = conv(x_flat, [dfa_mats(d, 4), dfa_mats(d, 5)],
                    [None, None], [False, False])
        (v,) = conv(k, [dfa_mats(d, 6)], [None], [False])
        e3 = (q * k).reshape(H, W8, LANES)
        m = fold32(jnp.max(e3, axis=1), jnp.maximum)
        e = jnp.exp(e3 - tile8(m)[:, None, :])
        s = jnp.dot(jnp.sum(e, axis=1), afold_ref[...],
                    preferred_element_type=jnp.float32)
        attn = e * pl.reciprocal(s, approx=False)[:, None, :]
        out = gamma_ref[d] * (v.reshape(H, W8, LANES) * attn)
        return out.reshape(HG, LANES)

    x0 = x_ref[0].reshape(HG, LANES)
    res = jax.lax.fori_loop(0, n_dfa, dfa_block, x0)
    tmats = ([tm_ref[0, 0, ky] for ky in range(3)],
             [te_ref[0, 0, ky] for ky in range(3)])
    (tail,) = conv(res, [tmats], [tbt_ref[...]], [False])
    o_ref[0] = (tail + x0).reshape(H, W8, LANES).astype(o_ref.dtype)


def kernel(x, w, b, caw1, cab1, caw2, cab2, tw, tb, gamma):
    N, H, W, C_ = x.shape
    assert C_ == C and W % P == 0
    W8 = W // P
    n_dfa = w.shape[0]
    CR = caw1.shape[-1] // 2
    cdt = jnp.bfloat16

    wm, we = _pack_conv(w, cdt)
    tm, te = _pack_conv(tw[None, ...], cdt)
    bt = jnp.tile(b.reshape(n_dfa, 7, 1, 1, C)[:, :4], (1, 1, 1, P, 1))
    bt = bt.reshape(n_dfa, 4, 1, LANES)
    tbt = jnp.tile(tb, (1, P))
    caw1t = jnp.tile(caw1, (1, P, 1))
    caw2t = jnp.tile(caw2.reshape(n_dfa, CR, 2, 1, C), (1, 1, 1, P, 1))
    caw2t = caw2t.reshape(n_dfa, CR, 2, LANES).transpose(0, 2, 1, 3)
    cab2t = jnp.tile(cab2.reshape(n_dfa, 2, 1, 1, C), (1, 1, 1, P, 1))
    cab2t = cab2t.reshape(n_dfa, 2, 1, LANES)
    afold = jnp.tile(jnp.eye(C, dtype=jnp.float32), (P, P))

    xp = x.reshape(N, H, W8, LANES)

    def full_spec(a):
        nd = a.ndim
        return pl.BlockSpec(a.shape, lambda n: (0,) * nd)

    _body = functools.partial(_dfag_kernel, H=H, W8=W8, CR=CR, n_dfa=n_dfa)

    out = pl.pallas_call(
        _body,
        out_shape=jax.ShapeDtypeStruct((N, H, W8, LANES), x.dtype),
        grid=(N,),
        in_specs=[
            pl.BlockSpec((1, H, W8, LANES), lambda n: (n, 0, 0, 0)),
            full_spec(wm), full_spec(we), full_spec(bt),
            full_spec(caw1t), full_spec(cab1), full_spec(caw2t),
            full_spec(cab2t), full_spec(afold),
            full_spec(tm), full_spec(te), full_spec(tbt),
            pl.BlockSpec(memory_space=pltpu.MemorySpace.SMEM),
        ],
        out_specs=pl.BlockSpec((1, H, W8, LANES), lambda n: (n, 0, 0, 0)),
        scratch_shapes=[pltpu.VMEM((H + 2, W0 + W8 + 2, LANES), cdt),
                        pltpu.VMEM((H + 2, W0 + W8 + 2, 2 * C), cdt)],
        compiler_params=pltpu.CompilerParams(
            dimension_semantics=("parallel",)),
    )(xp, wm, we, bt, caw1t, cab1, caw2t, cab2t, afold, tm, te, tbt, gamma)
    return out.reshape(N, H, W, C)


# speedup vs baseline: 4.5861x; 1.1931x over previous
import functools

import jax
import jax.numpy as jnp
from jax.experimental import pallas as pl
from jax.experimental.pallas import tpu as pltpu

P = 8
C = 32
LANES = P * C
W0 = 8


def _pack_conv(w, dtype):
    D = w.shape[0]
    S = w.shape[-1] // C
    w7 = w.reshape(D, 3, 3, C, S, C)
    w7 = jnp.transpose(w7, (0, 4, 1, 2, 3, 5))
    pi = jnp.arange(P)[:, None]
    po = jnp.arange(P)[None, :]
    sel = jnp.stack([(pi - po + 1 == t) for t in range(3)]).astype(w.dtype)
    main = jnp.einsum('tpq,dsytcf->dsypcqf', sel, w7)
    main = main.reshape(D, S, 3, P * C, P * C)
    zed = jnp.zeros((D, S, 3, C, P, C), w.dtype)
    left = zed.at[..., 0, :].set(w7[:, :, :, 0]).reshape(D, S, 3, C, P * C)
    right = zed.at[..., P - 1, :].set(w7[:, :, :, 2]).reshape(D, S, 3, C, P * C)
    edge = jnp.concatenate([left, right], axis=-2)
    return main.astype(dtype), edge.astype(dtype)


def _dfag_kernel(x_ref, wm_ref, we_ref, bt_ref,
                 caw1t_ref, cab1_ref, caw2t_ref, cab2t_ref, afold_ref,
                 tm_ref, te_ref, tbt_ref, gamma_ref,
                 o_ref, pad_ref, ec_ref, *, H, W8, CR, n_dfa):
    HG = H * W8
    HW = H * W8 * P

    pad_ref[...] = jnp.zeros_like(pad_ref)
    ec_ref[...] = jnp.zeros_like(ec_ref)

    cdt = wm_ref.dtype

    def conv(x_flat, mats, biases, relus):
        x3 = x_flat.reshape(H, W8, LANES)
        pad_ref[1:H + 1, W0:W0 + W8, :] = x3
        ec_ref[1:H + 1, W0 + 1:W0 + 1 + W8, :C] = x3[:, :, LANES - C:]
        ec_ref[1:H + 1, W0 - 1:W0 - 1 + W8, C:] = x3[:, :, :C]
        nout = len(mats)
        accs = [None] * nout
        for ky in range(3):
            pm = pad_ref[ky:ky + H, W0:W0 + W8, :].reshape(HG, LANES)
            pm = pm.astype(cdt)
            pe = ec_ref[ky:ky + H, W0:W0 + W8, :].reshape(HG, 2 * C)
            pe = pe.astype(cdt)
            for t, (m3, e3) in enumerate(mats):
                a = jnp.dot(pm, m3[ky], preferred_element_type=jnp.float32)
                a = a + jnp.dot(pe, e3[ky],
                                preferred_element_type=jnp.float32)
                accs[t] = a if accs[t] is None else accs[t] + a
        outs = []
        for acc, bias, relu in zip(accs, biases, relus):
            if bias is not None:
                acc = acc + bias
            if relu:
                acc = jnp.maximum(acc, 0.0)
            outs.append(acc)
        return outs

    def dfa_mats(d, s):
        return ([wm_ref[d, s, ky] for ky in range(3)],
                [we_ref[d, s, ky] for ky in range(3)])

    def fold32(v, op):
        v = op(v[:, :128], v[:, 128:])
        v = op(v[:, :64], v[:, 64:])
        return op(v[:, :32], v[:, 32:])

    def tile8(v):
        return jnp.concatenate([v] * P, axis=-1)

    def ca_layer(x_flat, d, r):
        pooled = jnp.sum(x_flat, axis=0, keepdims=True) * (1.0 / HW)
        w1 = caw1t_ref[d][:, r * CR:(r + 1) * CR]
        b1 = cab1_ref[d][:, r * CR:(r + 1) * CR]
        h = jnp.maximum(
            jnp.dot(pooled, w1, preferred_element_type=jnp.float32) + b1, 0.0)
        y = jax.nn.sigmoid(
            jnp.dot(h, caw2t_ref[d, r], preferred_element_type=jnp.float32)
            + cab2t_ref[d, r])
        return x_flat * y

    def rcab(x_flat, d, r):
        s1, s2 = 2 * r, 2 * r + 1
        (h1,) = conv(x_flat, [dfa_mats(d, s1)], [bt_ref[d, s1]], [True])
        (h2,) = conv(h1, [dfa_mats(d, s2)], [bt_ref[d, s2]], [False])
        return ca_layer(h2, d, r) + x_flat

    def dfa_block(d, x_flat):
        x_flat = rcab(x_flat, d, 0)
        x_flat = rcab(x_flat, d, 1)
        q, k = conv(x_flat, [dfa_mats(d, 4), dfa_mats(d, 5)],
                    [None, None], [False, False])
        (v,) = conv(k, [dfa_mats(d, 6)], [None], [False])
        e3 = (q * k).reshape(H, W8, LANES)
        m = fold32(jnp.max(e3, axis=1), jnp.maximum)
        e = jnp.exp(e3 - tile8(m)[:, None, :])
        s = jnp.dot(jnp.sum(e, axis=1), afold_ref[...],
                    preferred_element_type=jnp.float32)
        attn = e * pl.reciprocal(s, approx=False)[:, None, :]
        out = gamma_ref[d] * (v.reshape(H, W8, LANES) * attn)
        return out.reshape(HG, LANES)

    x0 = x_ref[0].reshape(HG, LANES)
    res = jax.lax.fori_loop(0, n_dfa, dfa_block, x0)
    tmats = ([tm_ref[0, 0, ky] for ky in range(3)],
             [te_ref[0, 0, ky] for ky in range(3)])
    (tail,) = conv(res, [tmats], [tbt_ref[...]], [False])
    o_ref[0] = (tail + x0).reshape(H, W8, LANES).astype(o_ref.dtype)


def kernel(x, w, b, caw1, cab1, caw2, cab2, tw, tb, gamma):
    N, H, W, C_ = x.shape
    assert C_ == C and W % P == 0
    W8 = W // P
    n_dfa = w.shape[0]
    CR = caw1.shape[-1] // 2
    cdt = jnp.bfloat16

    wm, we = _pack_conv(w, cdt)
    tm, te = _pack_conv(tw[None, ...], cdt)
    bt = jnp.tile(b.reshape(n_dfa, 7, 1, 1, C)[:, :4], (1, 1, 1, P, 1))
    bt = bt.reshape(n_dfa, 4, 1, LANES)
    tbt = jnp.tile(tb, (1, P))
    caw1t = jnp.tile(caw1, (1, P, 1))
    caw2t = jnp.tile(caw2.reshape(n_dfa, CR, 2, 1, C), (1, 1, 1, P, 1))
    caw2t = caw2t.reshape(n_dfa, CR, 2, LANES).transpose(0, 2, 1, 3)
    cab2t = jnp.tile(cab2.reshape(n_dfa, 2, 1, 1, C), (1, 1, 1, P, 1))
    cab2t = cab2t.reshape(n_dfa, 2, 1, LANES)
    afold = jnp.tile(jnp.eye(C, dtype=jnp.float32), (P, P))

    xp = x.reshape(N, H, W8, LANES)

    def full_spec(a):
        nd = a.ndim
        return pl.BlockSpec(a.shape, lambda n: (0,) * nd)

    _body = functools.partial(_dfag_kernel, H=H, W8=W8, CR=CR, n_dfa=n_dfa)

    out = pl.pallas_call(
        _body,
        out_shape=jax.ShapeDtypeStruct((N, H, W8, LANES), x.dtype),
        grid=(N,),
        in_specs=[
            pl.BlockSpec((1, H, W8, LANES), lambda n: (n, 0, 0, 0)),
            full_spec(wm), full_spec(we), full_spec(bt),
            full_spec(caw1t), full_spec(cab1), full_spec(caw2t),
            full_spec(cab2t), full_spec(afold),
            full_spec(tm), full_spec(te), full_spec(tbt),
            pl.BlockSpec(memory_space=pltpu.MemorySpace.SMEM),
        ],
        out_specs=pl.BlockSpec((1, H, W8, LANES), lambda n: (n, 0, 0, 0)),
        scratch_shapes=[pltpu.VMEM((H + 2, 24, LANES), jnp.float32),
                        pltpu.VMEM((H + 2, 24, 2 * C), jnp.float32)],
        compiler_params=pltpu.CompilerParams(
            dimension_semantics=("parallel",)),
    )(xp, wm, we, bt, caw1t, cab1, caw2t, cab2t, afold, tm, te, tbt, gamma)
    return out.reshape(N, H, W, C)
